# baseline (speedup 1.0000x reference)
import functools

import jax
import jax.numpy as jnp
from jax import lax
from jax.experimental import pallas as pl
from jax.experimental.pallas import tpu as pltpu
from jax.experimental.pallas import tpu_sc as plsc

SEQ_NUM = 1000000
N_EMBD = 64
WIN_LEN = 200
BATCH = 4096
INPUT_DIM = 64
OUT_DIM = INPUT_DIM + N_EMBD

_info = plsc.get_sparse_core_info()
_NC, _NS, _L = _info.num_cores, _info.num_subcores, _info.num_lanes
_NW = _NC * _NS
_WIN = 8


def _sc_gather(idx, tableT):
    n = idx.shape[0]
    b_per_w = n // _NW
    mesh = plsc.VectorSubcoreMesh(core_axis_name="c", subcore_axis_name="s")

    @functools.partial(
        pl.kernel,
        mesh=mesh,
        out_type=jax.ShapeDtypeStruct((n, N_EMBD), jnp.float32),
        scratch_types=[
            pltpu.VMEM((b_per_w,), jnp.int32),
            pltpu.VMEM((b_per_w, N_EMBD), jnp.float32),
            pltpu.VMEM((_WIN, N_EMBD, 128), jnp.float32),
            pltpu.SemaphoreType.DMA,
        ],
        compiler_params=pltpu.CompilerParams(needs_layout_passes=False),
    )
    def k(idx_hbm, t_hbm, out_hbm, idx_v, rows_v, tiles_v, sem):
        wid = lax.axis_index("s") * _NC + lax.axis_index("c")
        base = wid * b_per_w
        pltpu.sync_copy(idx_hbm.at[pl.ds(base, b_per_w)], idx_v)

        n_grp = b_per_w // _L
        ci = lax.iota(jnp.int32, _L)

        def issue(slot, r):
            rt = pl.multiple_of((r >> 7) << 7, 128)
            pltpu.async_copy(t_hbm.at[:, pl.ds(rt, 128)], tiles_v.at[slot], sem)

        def drain(slot):
            pltpu.make_async_copy(
                t_hbm.at[:, pl.ds(0, 128)], tiles_v.at[slot], sem
            ).wait()

        def extract(slot, r, j):
            rl = jnp.broadcast_to(r & 127, (_L,))
            for cg in range(N_EMBD // _L):
                res = plsc.load_gather(tiles_v.at[slot], [cg * _L + ci, rl])
                rows_v[j, pl.ds(cg * _L, _L)] = res

        def group(gi, carry):
            vec = idx_v[pl.ds(gi * _L, _L)]
            for kk in range(_WIN):
                issue(kk, vec[kk])
            for kk in range(_WIN, _L):
                s = kk - _WIN
                drain(s)
                extract(s, vec[s], gi * _L + s)
                issue(s, vec[kk])
            for kk in range(_L - _WIN, _L):
                s = kk - (_L - _WIN)
                drain(s)
                extract(s, vec[kk], gi * _L + kk)
            return carry

        lax.fori_loop(0, n_grp, group, 0)
        pltpu.sync_copy(rows_v, out_hbm.at[pl.ds(base, b_per_w)])

    return k(idx, tableT)


_BBT = 128
_NBT = BATCH // _BBT
_OUT_SHAPE = jax.ShapeDtypeStruct((BATCH, WIN_LEN, OUT_DIM), jnp.float32)


_SPLIT = 1024


def _tc_body(x_ref, g_ref, po_ref, o_ref):
    xv = x_ref[...]
    o_ref[:, :, 0:INPUT_DIM] = jnp.transpose(xv, (2, 0, 1))
    o_ref[:, :, INPUT_DIM:] = g_ref[...][:, None, :] + po_ref[...][None, :, :]


def _tc_concat_part(xT, g_part, po_table, row0, prev=None):
    base = row0 // _BBT
    steps = g_part.shape[0] // _BBT

    def _body(*refs):
        if prev is None:
            _tc_body(*refs)
        else:
            _tc_body(*refs[1:])

    in_specs = [
        pl.BlockSpec((WIN_LEN, INPUT_DIM, _BBT), lambda i: (0, 0, base + i)),
        pl.BlockSpec((_BBT, N_EMBD), lambda i: (i, 0)),
        pl.BlockSpec((WIN_LEN, N_EMBD), lambda i: (0, 0)),
    ]
    args = (xT, g_part, po_table)
    kwargs = {}
    if prev is not None:
        in_specs = [pl.BlockSpec(memory_space=pl.ANY)] + in_specs
        args = (prev,) + args
        kwargs = dict(input_output_aliases={0: 0})
    return pl.pallas_call(
        _body,
        grid=(steps,),
        in_specs=in_specs,
        out_specs=pl.BlockSpec((_BBT, WIN_LEN, OUT_DIM), lambda i: (base + i, 0, 0)),
        out_shape=_OUT_SHAPE,
        **kwargs,
    )(*args)


@jax.jit
def kernel(series_id, x, id_table, po_table):
    sid = series_id.astype(jnp.int32)
    tableT = id_table.T
    xT = jnp.transpose(x, (1, 2, 0))
    g0 = _sc_gather(sid[:_SPLIT], tableT)
    g1 = _sc_gather(sid[_SPLIT:], tableT)
    out0 = _tc_concat_part(xT, g0, po_table, 0)
    return _tc_concat_part(xT, g1, po_table, _SPLIT, prev=out0)
